# baseline (speedup 1.0000x reference)
import jax
import jax.numpy as jnp
from jax.experimental import pallas as pl

_ALPHA = 0.1
_BETA = 0.5


def _fused_body(g_ref, f_ref, f0_ref, w_ref, b_ref, o_ref):
    g = g_ref[...].astype(jnp.bfloat16)
    hi = jnp.dot(g, f_ref[...], preferred_element_type=jnp.float32)
    support = (1.0 - _ALPHA) * hi + _ALPHA * f0_ref[...]
    out = _BETA * jnp.dot(support, w_ref[...], preferred_element_type=jnp.float32)
    o_ref[...] = out + (1.0 - _BETA) * support + b_ref[...]


def kernel(graph, features, features0, w, b):
    n, k = graph.shape
    f = features.shape[1]
    fo = w.shape[1]
    b2 = b.reshape(1, fo)
    features = features.astype(jnp.bfloat16)

    bm = 400 if n % 400 == 0 else n
    grid = (n // bm,)

    return pl.pallas_call(
        _fused_body,
        grid=grid,
        in_specs=[
            pl.BlockSpec((bm, k), lambda i: (i, 0)),
            pl.BlockSpec((k, f), lambda i: (0, 0)),
            pl.BlockSpec((bm, f), lambda i: (i, 0)),
            pl.BlockSpec((f, fo), lambda i: (0, 0)),
            pl.BlockSpec((1, fo), lambda i: (0, 0)),
        ],
        out_specs=pl.BlockSpec((bm, fo), lambda i: (i, 0)),
        out_shape=jax.ShapeDtypeStruct((n, fo), jnp.float32),
    )(graph, features, features0, w, b2)


# speedup vs baseline: 1.0349x; 1.0349x over previous
import jax
import jax.numpy as jnp
from jax.experimental import pallas as pl
from jax.experimental.pallas import tpu as pltpu

_ALPHA = 0.1
_BETA = 0.5


def _fused_body(g_ref, f_ref, f0_ref, w_ref, b_ref, o_ref):
    hi = jnp.dot(g_ref[...], f_ref[...], preferred_element_type=jnp.float32)
    support = (1.0 - _ALPHA) * hi + _ALPHA * f0_ref[...]
    out = _BETA * jnp.dot(support, w_ref[...], preferred_element_type=jnp.float32)
    o_ref[...] = out + (1.0 - _BETA) * support + b_ref[...]


def kernel(graph, features, features0, w, b):
    n, k = graph.shape
    f = features.shape[1]
    fo = w.shape[1]
    b2 = b.reshape(1, fo)

    bm = 400 if n % 400 == 0 else n
    grid = (n // bm,)

    return pl.pallas_call(
        _fused_body,
        grid=grid,
        in_specs=[
            pl.BlockSpec((bm, k), lambda i: (i, 0)),
            pl.BlockSpec((k, f), lambda i: (0, 0)),
            pl.BlockSpec((bm, f), lambda i: (i, 0)),
            pl.BlockSpec((f, fo), lambda i: (0, 0)),
            pl.BlockSpec((1, fo), lambda i: (0, 0)),
        ],
        out_specs=pl.BlockSpec((bm, fo), lambda i: (i, 0)),
        out_shape=jax.ShapeDtypeStruct((n, fo), jnp.float32),
        compiler_params=pltpu.CompilerParams(
            dimension_semantics=("parallel",),
        ),
    )(graph, features, features0, w, b2)


# speedup vs baseline: 1.0362x; 1.0013x over previous
import jax
import jax.numpy as jnp
from jax.experimental import pallas as pl
from jax.experimental.pallas import tpu as pltpu

_ALPHA = 0.1
_BETA = 0.5


def _fused_body(g_ref, f_ref, f0_ref, w_ref, b_ref, o_ref):
    hi = jnp.dot(g_ref[...], f_ref[...], preferred_element_type=jnp.float32)
    support = (1.0 - _ALPHA) * hi + _ALPHA * f0_ref[...]
    out = _BETA * jnp.dot(support, w_ref[...], preferred_element_type=jnp.float32)
    o_ref[...] = out + (1.0 - _BETA) * support + b_ref[...]


def kernel(graph, features, features0, w, b):
    n, k = graph.shape
    f = features.shape[1]
    fo = w.shape[1]
    b2 = b.reshape(1, fo)

    bm = 200 if n % 200 == 0 else n
    grid = (n // bm,)

    return pl.pallas_call(
        _fused_body,
        grid=grid,
        in_specs=[
            pl.BlockSpec((bm, k), lambda i: (i, 0)),
            pl.BlockSpec((k, f), lambda i: (0, 0)),
            pl.BlockSpec((bm, f), lambda i: (i, 0)),
            pl.BlockSpec((f, fo), lambda i: (0, 0)),
            pl.BlockSpec((1, fo), lambda i: (0, 0)),
        ],
        out_specs=pl.BlockSpec((bm, fo), lambda i: (i, 0)),
        out_shape=jax.ShapeDtypeStruct((n, fo), jnp.float32),
        compiler_params=pltpu.CompilerParams(
            dimension_semantics=("parallel",),
        ),
    )(graph, features, features0, w, b2)
